# baseline (speedup 1.0000x reference)
import functools

import jax
import jax.numpy as jnp
from jax import lax
from jax.experimental import pallas as pl
from jax.experimental.pallas import tpu as pltpu
from jax.experimental.pallas import tpu_sc as plsc

VOCAB = 1000000
BATCH = 16384
UNITS = 10

_NC = 2
_NS = 16
_NW = _NC * _NS
_BPW = BATCH // _NW
_ICH = 128
_KCH = _BPW // _ICH
_L = 16
_CPG = _ICH // _L
_OPW = _BPW * UNITS

_IB = lax.GatherScatterMode.PROMISE_IN_BOUNDS

_DNUMS = lax.GatherDimensionNumbers(
    offset_dims=(), collapsed_slice_dims=(0,), start_index_map=(0,)
)


def _lanegather(v, pat):
    return lax.gather(
        v, pat[:, None], dimension_numbers=_DNUMS, slice_sizes=(1,), mode=_IB
    )


def _lanepats():
    l16 = lax.iota(jnp.int32, 16)
    rows, cols = [], []
    for m in range(UNITS):
        n = l16 + (16 * m)
        row = lax.shift_right_logical(n * 6554, 16)
        rows.append(row)
        cols.append(n - row * 10)
    return rows, cols


_mesh = plsc.VectorSubcoreMesh(
    core_axis_name="c", subcore_axis_name="s", num_cores=_NC, num_subcores=_NS
)


@functools.partial(
    pl.kernel,
    out_type=jax.ShapeDtypeStruct((_NW, _OPW), jnp.float32),
    mesh=_mesh,
    scratch_types=[
        pltpu.VMEM((_KCH, _ICH), jnp.int32),
        pltpu.VMEM((_BPW,), jnp.int32),
        pltpu.VMEM((2, _L), jnp.float32),
        pltpu.VMEM((_OPW,), jnp.float32),
        pltpu.SemaphoreType.DMA,
        pltpu.SemaphoreType.DMA,
        pltpu.SemaphoreType.DMA,
        pltpu.SemaphoreType.DMA,
        pltpu.SemaphoreType.DMA,
    ],
)
def _lookup_affine(
    table_h, idx_h, w_h, b_h, out_h, idx_v, ids_v, wb_v, out_v, wb_sem, *sems
):
    wid = lax.axis_index("s") * _NC + lax.axis_index("c")
    wcp = pltpu.async_copy(w_h, wb_v.at[0, pl.ds(0, UNITS)], wb_sem)
    bcp = pltpu.async_copy(b_h, wb_v.at[1, pl.ds(0, UNITS)], wb_sem)
    pltpu.sync_copy(idx_h.at[wid], idx_v)
    copies = [
        pltpu.async_copy(
            table_h.at[idx_v.at[k]], ids_v.at[pl.ds(k * _ICH, _ICH)], sems[k]
        )
        for k in range(_KCH)
    ]
    wcp.wait()
    bcp.wait()
    rowpat, colpat = _lanepats()
    wv = [_lanegather(wb_v[0], colpat[m]) for m in range(UNITS)]
    bv = [_lanegather(wb_v[1], colpat[m]) for m in range(UNITS)]
    for k in range(_KCH):
        copies[k].wait()
        for cc in range(_CPG):
            c = k * _CPG + cc
            idf = ids_v[pl.ds(c * _L, _L)].astype(jnp.float32)
            for m in range(UNITS):
                out_v[pl.ds(c * UNITS * _L + m * _L, _L)] = (
                    _lanegather(idf, rowpat[m]) * wv[m] + bv[m]
                )
    pltpu.sync_copy(out_v, out_h.at[wid])


def kernel(inputs, lookup_table, W, b):
    idx = inputs.reshape(-1).astype(jnp.int32).reshape(_NW, _KCH, _ICH)
    out = _lookup_affine(
        lookup_table, idx, W.reshape(UNITS).astype(jnp.float32),
        b.astype(jnp.float32)
    )
    return out.reshape(BATCH, UNITS)
